# baseline (speedup 1.0000x reference)
import functools

import jax
import jax.numpy as jnp
from jax import lax
from jax.experimental import pallas as pl
from jax.experimental.pallas import tpu as pltpu
from jax.experimental.pallas import tpu_sc as plsc

B, L, D = 4, 8192, 1024
NC, NS = 2, 16
NW = NC * NS
ROWS_PER_W = L // NW
R = 32
NCHUNK = ROWS_PER_W // R
CH = R * D
GROUPS = CH // 16


@functools.partial(
    pl.kernel,
    out_type=jax.ShapeDtypeStruct((B * L * D,), jnp.float32),
    mesh=plsc.VectorSubcoreMesh(core_axis_name="c", subcore_axis_name="s"),
    scratch_types=[
        pltpu.VMEM((CH,), jnp.float32),
        pltpu.VMEM((CH,), jnp.float32),
    ],
)
def _sc_add(x_hbm, pos_hbm, out_hbm, posb, xb):
    wid = lax.axis_index("s") * NC + lax.axis_index("c")
    base = wid * (ROWS_PER_W * D)

    def chunk_body(j, carry):
        off = base + j * CH
        pltpu.sync_copy(pos_hbm.at[pl.ds(off, CH)], posb)
        for b in range(B):
            boff = b * (L * D) + off
            pltpu.sync_copy(x_hbm.at[pl.ds(boff, CH)], xb)

            def grp(i, c):
                sl = pl.ds(i * 16, 16)
                xb[sl] = xb[sl] + posb[sl]
                return c

            lax.fori_loop(0, GROUPS, grp, 0, unroll=8)
            pltpu.sync_copy(xb, out_hbm.at[pl.ds(boff, CH)])
        return carry

    lax.fori_loop(0, NCHUNK, chunk_body, 0)


def kernel(x, pos_emb):
    b, l, d = x.shape
    out = _sc_add(x.reshape(-1), pos_emb[:l].reshape(-1))
    return out.reshape(b, l, d)


# speedup vs baseline: 1.2004x; 1.2004x over previous
import functools

import jax
import jax.numpy as jnp
from jax import lax
from jax.experimental import pallas as pl
from jax.experimental.pallas import tpu as pltpu
from jax.experimental.pallas import tpu_sc as plsc

B, L, D = 4, 8192, 1024
NC, NS = 2, 16
NW = NC * NS
ROWS_PER_W = L // NW
R = 8
NCHUNK = ROWS_PER_W // R
NPAIR = NCHUNK // 2
CH = R * D
GROUPS = CH // 16
NSLOT = 8

_scratch = (
    [pltpu.VMEM((CH,), jnp.float32) for _ in range(NSLOT)]
    + [pltpu.VMEM((CH,), jnp.float32) for _ in range(2)]
    + [pltpu.SemaphoreType.DMA for _ in range(NSLOT + 2)]
)


@functools.partial(
    pl.kernel,
    out_type=jax.ShapeDtypeStruct((B * L * D,), jnp.float32),
    mesh=plsc.VectorSubcoreMesh(core_axis_name="c", subcore_axis_name="s"),
    scratch_types=_scratch,
)
def _sc_add(x_hbm, pos_hbm, out_hbm, *refs):
    xbs = refs[:NSLOT]
    pbs = refs[NSLOT:NSLOT + 2]
    sxs = refs[NSLOT + 2:NSLOT + 2 + NSLOT]
    sps = refs[NSLOT + 2 + NSLOT:]

    wid = lax.axis_index("s") * NC + lax.axis_index("c")
    base = wid * (ROWS_PER_W * D)

    def start_in(j, b, slot):
        src = x_hbm.at[pl.ds(b * (L * D) + base + j * CH, CH)]
        pltpu.async_copy(src, xbs[slot], sxs[slot])

    def wait_in(slot):
        pltpu.make_async_copy(
            x_hbm.at[pl.ds(0, CH)], xbs[slot], sxs[slot]).wait()

    def start_out(j, b, slot):
        dst = out_hbm.at[pl.ds(b * (L * D) + base + j * CH, CH)]
        pltpu.async_copy(xbs[slot], dst, sxs[slot])

    def wait_out(slot):
        pltpu.make_async_copy(
            xbs[slot], out_hbm.at[pl.ds(0, CH)], sxs[slot]).wait()

    def start_pos(j, pslot):
        pltpu.async_copy(
            pos_hbm.at[pl.ds(base + j * CH, CH)], pbs[pslot], sps[pslot])

    def wait_pos(pslot):
        pltpu.make_async_copy(
            pos_hbm.at[pl.ds(0, CH)], pbs[pslot], sps[pslot]).wait()

    def add_chunk(slot, pslot):
        xb, pb = xbs[slot], pbs[pslot]

        def grp(i, c):
            sl = pl.ds(i * 16, 16)
            xb[sl] = xb[sl] + pb[sl]
            return c

        lax.fori_loop(0, GROUPS, grp, 0, unroll=8)

    start_pos(0, 0)
    start_pos(1, 1)
    for b in range(B):
        start_in(0, b, b)

    def pair_body(jj, carry):
        for half in range(2):
            j = 2 * jj + half
            ha = half * 4
            hb = (1 - half) * 4
            wait_pos(half)
            for b in range(B):
                s_cur = ha + b
                s_nxt = hb + b
                if half == 0:
                    @pl.when(jj > 0)
                    def _():
                        wait_out(s_nxt)
                    start_in(j + 1, b, s_nxt)
                else:
                    wait_out(s_nxt)

                    @pl.when(jj < NPAIR - 1)
                    def _():
                        start_in(j + 1, b, s_nxt)
                wait_in(s_cur)
                add_chunk(s_cur, half)
                start_out(j, b, s_cur)

            @pl.when(jj < NPAIR - 1)
            def _():
                start_pos(j + 2, half)
        return carry

    lax.fori_loop(0, NPAIR, pair_body, 0)

    for b in range(B):
        wait_out(4 + b)


def kernel(x, pos_emb):
    b, l, d = x.shape
    out = _sc_add(x.reshape(-1), pos_emb[:l].reshape(-1))
    return out.reshape(b, l, d)


# speedup vs baseline: 2.1082x; 1.7562x over previous
import functools

import jax
import jax.numpy as jnp
from jax import lax
from jax.experimental import pallas as pl
from jax.experimental.pallas import tpu as pltpu
from jax.experimental.pallas import tpu_sc as plsc

B, L, D = 4, 8192, 1024
NC, NS = 2, 16
NW = NC * NS
ROWS_PER_W = L // NW
R = 8
NCHUNK = ROWS_PER_W // R
NPAIR = NCHUNK // 2
GROUPS = D // 16
NSLOT = 8

_scratch = (
    [pltpu.VMEM((R, D), jnp.float32) for _ in range(NSLOT)]
    + [pltpu.VMEM((R, D), jnp.float32) for _ in range(2)]
    + [pltpu.SemaphoreType.DMA for _ in range(NSLOT + 2)]
)


@functools.partial(
    pl.kernel,
    out_type=jax.ShapeDtypeStruct((B, L, D), jnp.float32),
    mesh=plsc.VectorSubcoreMesh(core_axis_name="c", subcore_axis_name="s"),
    scratch_types=_scratch,
)
def _sc_add(x_hbm, pos_hbm, out_hbm, *refs):
    xbs = refs[:NSLOT]
    pbs = refs[NSLOT:NSLOT + 2]
    sxs = refs[NSLOT + 2:NSLOT + 2 + NSLOT]
    sps = refs[NSLOT + 2 + NSLOT:]

    wid = lax.axis_index("s") * NC + lax.axis_index("c")
    row0 = wid * ROWS_PER_W

    def start_in(j, b, slot):
        src = x_hbm.at[b, pl.ds(row0 + j * R, R), :]
        pltpu.async_copy(src, xbs[slot], sxs[slot])

    def wait_in(slot):
        pltpu.make_async_copy(
            x_hbm.at[0, pl.ds(0, R), :], xbs[slot], sxs[slot]).wait()

    def start_out(j, b, slot):
        dst = out_hbm.at[b, pl.ds(row0 + j * R, R), :]
        pltpu.async_copy(xbs[slot], dst, sxs[slot])

    def wait_out(slot):
        pltpu.make_async_copy(
            xbs[slot], out_hbm.at[0, pl.ds(0, R), :], sxs[slot]).wait()

    def start_pos(j, pslot):
        pltpu.async_copy(
            pos_hbm.at[pl.ds(row0 + j * R, R), :], pbs[pslot], sps[pslot])

    def wait_pos(pslot):
        pltpu.make_async_copy(
            pos_hbm.at[pl.ds(0, R), :], pbs[pslot], sps[pslot]).wait()

    def add_chunk(slot, pslot):
        xb, pb = xbs[slot], pbs[pslot]

        def row(r, c):
            def grp(i, c2):
                sl = pl.ds(i * 16, 16)
                xb[r, sl] = xb[r, sl] + pb[r, sl]
                return c2

            return lax.fori_loop(0, GROUPS, grp, c, unroll=8)

        lax.fori_loop(0, R, row, 0)

    def pair_iter(jj, first=False, last=False):
        for half in range(2):
            j = 2 * jj + half
            ha = half * 4
            hb = (1 - half) * 4
            wait_pos(half)
            for b in range(B):
                s_cur = ha + b
                s_nxt = hb + b
                if not (first and half == 0):
                    wait_out(s_nxt)
                if not (last and half == 1):
                    start_in(j + 1, b, s_nxt)
                wait_in(s_cur)
                add_chunk(s_cur, half)
                start_out(j, b, s_cur)
            if not last:
                start_pos(j + 2, half)

    start_pos(0, 0)
    start_pos(1, 1)
    for b in range(B):
        start_in(0, b, b)

    pair_iter(0, first=True)

    def pair_body(jj, carry):
        pair_iter(jj)
        return carry

    lax.fori_loop(1, NPAIR - 1, pair_body, 0)

    pair_iter(NPAIR - 1, last=True)

    for b in range(B):
        wait_out(4 + b)


def kernel(x, pos_emb):
    b, l, d = x.shape
    return _sc_add(x, pos_emb[:l])


# speedup vs baseline: 5.6373x; 2.6740x over previous
import functools

import jax
import jax.numpy as jnp
from jax import lax
from jax.experimental import pallas as pl
from jax.experimental.pallas import tpu as pltpu
from jax.experimental.pallas import tpu_sc as plsc

B, L, D = 4, 8192, 1024
NC, NS = 2, 16
NW = NC * NS
ROWS_PER_W = L // NW
R = 8
NCHUNK = ROWS_PER_W // R
NPAIR = NCHUNK // 2
GROUPS = D // 16
NSLOT = 8

_scratch = (
    [pltpu.VMEM((R, D), jnp.float32) for _ in range(NSLOT)]
    + [pltpu.VMEM((R, D), jnp.float32) for _ in range(2)]
    + [pltpu.SemaphoreType.DMA for _ in range(NSLOT + 2)]
)


@functools.partial(
    pl.kernel,
    out_type=jax.ShapeDtypeStruct((B, L, D), jnp.float32),
    mesh=plsc.VectorSubcoreMesh(core_axis_name="c", subcore_axis_name="s"),
    scratch_types=_scratch,
)
def _sc_add(x_hbm, pos_hbm, out_hbm, *refs):
    xbs = refs[:NSLOT]
    pbs = refs[NSLOT:NSLOT + 2]
    sxs = refs[NSLOT + 2:NSLOT + 2 + NSLOT]
    sps = refs[NSLOT + 2 + NSLOT:]

    wid = lax.axis_index("s") * NC + lax.axis_index("c")
    row0 = wid * ROWS_PER_W

    def start_in(j, b, slot):
        src = x_hbm.at[b, pl.ds(row0 + j * R, R), :]
        pltpu.async_copy(src, xbs[slot], sxs[slot])

    def wait_in(slot):
        pltpu.make_async_copy(
            x_hbm.at[0, pl.ds(0, R), :], xbs[slot], sxs[slot]).wait()

    def start_out(j, b, slot):
        dst = out_hbm.at[b, pl.ds(row0 + j * R, R), :]
        pltpu.async_copy(xbs[slot], dst, sxs[slot])

    def wait_out(slot):
        pltpu.make_async_copy(
            xbs[slot], out_hbm.at[0, pl.ds(0, R), :], sxs[slot]).wait()

    def start_pos(j, pslot):
        pltpu.async_copy(
            pos_hbm.at[pl.ds(row0 + j * R, R), :], pbs[pslot], sps[pslot])

    def wait_pos(pslot):
        pltpu.make_async_copy(
            pos_hbm.at[pl.ds(0, R), :], pbs[pslot], sps[pslot]).wait()

    def add_chunk(slot, pslot):
        xb, pb = xbs[slot], pbs[pslot]

        def row(r, c):
            @plsc.parallel_loop(0, GROUPS, unroll=8)
            def grp(i):
                sl = pl.ds(i * 16, 16)
                xb[r, sl] = xb[r, sl] + pb[r, sl]

            return c

        lax.fori_loop(0, R, row, 0)

    def pair_iter(jj, first=False, last=False):
        for half in range(2):
            j = 2 * jj + half
            ha = half * 4
            hb = (1 - half) * 4
            wait_pos(half)
            for b in range(B):
                s_cur = ha + b
                s_nxt = hb + b
                if not (first and half == 0):
                    wait_out(s_nxt)
                if not (last and half == 1):
                    start_in(j + 1, b, s_nxt)
                wait_in(s_cur)
                add_chunk(s_cur, half)
                start_out(j, b, s_cur)
            if not last:
                start_pos(j + 2, half)

    start_pos(0, 0)
    start_pos(1, 1)
    for b in range(B):
        start_in(0, b, b)

    pair_iter(0, first=True)

    def pair_body(jj, carry):
        pair_iter(jj)
        return carry

    lax.fori_loop(1, NPAIR - 1, pair_body, 0)

    pair_iter(NPAIR - 1, last=True)

    for b in range(B):
        wait_out(4 + b)


def kernel(x, pos_emb):
    b, l, d = x.shape
    return _sc_add(x, pos_emb[:l])
